# baseline (speedup 1.0000x reference)
import functools

import jax
import jax.numpy as jnp
from jax import lax
from jax.experimental import pallas as pl
from jax.experimental.pallas import tpu as pltpu
from jax.experimental.pallas import tpu_sc as plsc


def _make_sc_copy(b, s, d, dtype):
    info = plsc.get_sparse_core_info()
    nw = info.num_cores * info.num_subcores
    rows_per_w = s // nw
    ch = 32
    n_ch = rows_per_w // ch
    mesh = plsc.VectorSubcoreMesh(core_axis_name="c", subcore_axis_name="s")

    @functools.partial(
        pl.kernel,
        mesh=mesh,
        out_type=jax.ShapeDtypeStruct((b * s, d), dtype),
        scratch_types=[
            pltpu.VMEM((ch, d), dtype),
            pltpu.VMEM((ch, d), dtype),
            pltpu.SemaphoreType.DMA,
            pltpu.SemaphoreType.DMA,
            pltpu.SemaphoreType.DMA,
            pltpu.SemaphoreType.DMA,
        ],
    )
    def sc_copy(emb_hbm, out_hbm, buf0, buf1, rsem0, rsem1, wsem0, wsem1):
        wid = lax.axis_index("s") * info.num_cores + lax.axis_index("c")
        base = wid * rows_per_w
        bufs = (buf0, buf1)
        rsems = (rsem0, rsem1)
        wsems = (wsem0, wsem1)
        reads = [None, None]
        writes = [[], []]
        reads[0] = pltpu.async_copy(emb_hbm.at[pl.ds(base, ch)], buf0, rsem0)
        for c in range(n_ch):
            cur = c % 2
            nxt = (c + 1) % 2
            if c + 1 < n_ch:
                for wcp in writes[nxt]:
                    wcp.wait()
                writes[nxt] = []
                reads[nxt] = pltpu.async_copy(
                    emb_hbm.at[pl.ds(base + (c + 1) * ch, ch)], bufs[nxt],
                    rsems[nxt])
            reads[cur].wait()
            row0 = base + c * ch
            for bb in range(b):
                writes[cur].append(
                    pltpu.async_copy(bufs[cur],
                                     out_hbm.at[pl.ds(bb * s + row0, ch)],
                                     wsems[cur]))
        for side in writes:
            for wcp in side:
                wcp.wait()

    return sc_copy


def kernel(x, emb):
    b, s = x.shape
    max_seq_len, d = emb.shape
    assert s < max_seq_len
    out = _make_sc_copy(b, s, d, emb.dtype)(emb)
    return out.reshape(b, s, d)
